# baseline (speedup 1.0000x reference)
import functools

import jax
import jax.numpy as jnp
from jax import lax
from jax.experimental import pallas as pl
from jax.experimental.pallas import tpu as pltpu
from jax.experimental.pallas import tpu_sc as plsc

BATCH = 16384
NROWS = 1000000
EMBED_DIM = 64
LANES = 16
NUM_WORKERS = 32
STRIPE = 32768
CHUNK = 4096
NCHUNK = STRIPE // CHUNK
STAGE = 112
B_PER_W = BATCH // NUM_WORKERS
NVEC = BATCH // LANES

_MAIN_END = 983040
_W30_END = 999936
_TAIL = NROWS - _W30_END


def _gather_side(idx_hbm, idx_v, tab3, emb_hbm, lo, n_chunks, tail512,
                 tail64, slist_v, clist_v, chunk_v, chunk2_v, off_v,
                 stage_v, dsem, wsem):
    iota = lax.iota(jnp.int32, LANES)
    pltpu.sync_copy(idx_hbm, idx_v)
    hi = jnp.where(tail64, NROWS,
                   jnp.where(tail512, _W30_END, lo + n_chunks * CHUNK))

    def scan_body(v, cnt):
        vec = idx_v[pl.ds(v * LANES, LANES)]
        m = (vec >= lo) & (vec < hi)
        plsc.store_compressed(slist_v.at[pl.ds(cnt, LANES)], iota + v * LANES, mask=m)
        return cnt + plsc.all_reduce_population_count(m)[0]

    scnt = lax.fori_loop(0, NVEC, scan_body, jnp.int32(0))

    def process_chunk(cb, width):
        def cscan(j, cc):
            posv = slist_v[pl.ds(j * LANES, LANES)]
            mv = (iota + j * LANES) < scnt
            uv = plsc.load_gather(idx_v, [posv], mask=mv)
            m = mv & (uv >= cb) & (uv < cb + width)
            plsc.store_compressed(clist_v.at[pl.ds(cc, LANES)], posv, mask=m)
            return cc + plsc.all_reduce_population_count(m)[0]

        ccnt = lax.fori_loop(0, (scnt + LANES - 1) // LANES, cscan,
                             jnp.int32(0))

        def batch_body(b, _):
            bbase = b * STAGE
            cnt_b = jnp.minimum(ccnt - bbase, STAGE)
            ngroups = (cnt_b + LANES - 1) // LANES

            def ogroup(jj, _):
                ev = iota + jj * LANES
                mb = ev < cnt_b
                posv = clist_v[pl.ds(bbase + jj * LANES, LANES)]
                uv = plsc.load_gather(idx_v, [posv], mask=mb)
                off_v[pl.ds(jj * LANES, LANES)] = uv - cb
                return 0

            lax.fori_loop(0, ngroups, ogroup, 0)

            def gather_slab(s, buf):
                def ggroup(jj, _):
                    ev = iota + jj * LANES
                    mb = ev < cnt_b
                    off = off_v[pl.ds(jj * LANES, LANES)]
                    for k in range(8):
                        vals = plsc.load_gather(
                            buf, [jnp.full((LANES,), k, jnp.int32), off],
                            mask=mb)
                        plsc.store_scatter(
                            stage_v,
                            [ev, jnp.broadcast_to(s * 8 + k, (LANES,))],
                            vals, mask=mb)
                    return 0

                lax.fori_loop(0, ngroups, ggroup, 0)

            def issue(s, buf):
                return pltpu.async_copy(
                    tab3.at[s, :, pl.ds(cb, width)],
                    buf.at[:, pl.ds(0, width)], dsem)

            def wait_one():
                pltpu.make_async_copy(
                    tab3.at[0, :, pl.ds(cb, width)],
                    chunk_v.at[:, pl.ds(0, width)], dsem).wait()

            issue(0, chunk_v)

            def pair_body(t, _):
                issue(2 * t + 1, chunk2_v)
                wait_one()
                gather_slab(2 * t, chunk_v)

                @pl.when(t < 3)
                def _pref():
                    issue(2 * t + 2, chunk_v)

                wait_one()
                gather_slab(2 * t + 1, chunk2_v)
                return 0

            lax.fori_loop(0, 4, pair_body, 0)

            def fgroup(jj, _):
                ev = iota + jj * LANES
                posv = clist_v[pl.ds(bbase + jj * LANES, LANES)]
                posv = jnp.where(ev < cnt_b, posv, BATCH)
                for l in range(LANES):
                    pltpu.async_copy(
                        stage_v.at[pl.ds(jj * LANES + l, 1), :],
                        emb_hbm.at[pl.ds(posv[l], 1), :], wsem)
                return 0

            lax.fori_loop(0, ngroups, fgroup, 0)

            def fdrain(i, _):
                pltpu.make_async_copy(
                    stage_v.at[pl.ds(0, 1), :],
                    emb_hbm.at[pl.ds(BATCH, 1), :], wsem).wait()
                return 0

            lax.fori_loop(0, ngroups * LANES, fdrain, 0)
            return 0

        lax.fori_loop(0, (ccnt + STAGE - 1) // STAGE, batch_body, 0)

    def chunk_loop(c, _):
        cb = pl.multiple_of(lo + c * CHUNK, CHUNK)
        process_chunk(cb, CHUNK)
        return 0

    lax.fori_loop(0, n_chunks, chunk_loop, 0)

    @pl.when(tail512)
    def _t512():
        process_chunk(pl.multiple_of(jnp.int32(_MAIN_END + 4 * CHUNK), 512),
                      512)

    @pl.when(tail64)
    def _t64():
        process_chunk(pl.multiple_of(jnp.int32(_W30_END), 64), _TAIL)


def _gather_body(users_hbm, items_hbm, utab3, itab3, ue_hbm, ie_hbm,
                 idx_v, slist_v, clist_v, chunk_v, chunk2_v, off_v,
                 stage_v, isem, dsem, wsem):
    wid = lax.axis_index("s") * 2 + lax.axis_index("c")

    lo = wid * STRIPE
    n_chunks = jnp.where(wid < 30, NCHUNK, jnp.where(wid == 30, 4, 0))
    tail512 = wid == 30
    tail64 = wid == 31
    lo = jnp.where(tail64, _W30_END, lo)

    _gather_side(users_hbm, idx_v, utab3, ue_hbm, lo, n_chunks, tail512,
                 tail64, slist_v, clist_v, chunk_v, chunk2_v, off_v,
                 stage_v, dsem, wsem)
    _gather_side(items_hbm, idx_v, itab3, ie_hbm, lo, n_chunks, tail512,
                 tail64, slist_v, clist_v, chunk_v, chunk2_v, off_v,
                 stage_v, dsem, wsem)


def _dot_body(ue_hbm, ie_hbm, out_hbm, ubuf_v, ibuf_v, out_v, part_v, sem):
    wid = lax.axis_index("s") * 2 + lax.axis_index("c")
    base = wid * B_PER_W
    half = B_PER_W // 2
    lane_iota = lax.iota(jnp.int32, LANES)

    def do_half(h, _):
        hbase = base + h * half
        cp1 = pltpu.async_copy(ue_hbm.at[pl.ds(hbase, half), :], ubuf_v, sem)
        cp1.wait()
        cp2 = pltpu.async_copy(ie_hbm.at[pl.ds(hbase, half), :], ibuf_v, sem)
        cp2.wait()

        def group(g, carry):
            rowbase = g * LANES
            for r in range(LANES):
                row = rowbase + r
                acc = (ubuf_v[row, pl.ds(0, LANES)]
                       * ibuf_v[row, pl.ds(0, LANES)])
                for k in range(1, EMBED_DIM // LANES):
                    acc = acc + (ubuf_v[row, pl.ds(k * LANES, LANES)]
                                 * ibuf_v[row, pl.ds(k * LANES, LANES)])
                part_v[pl.ds(r * LANES, LANES)] = acc
            tot = plsc.load_gather(part_v, [lane_iota * LANES])
            for l in range(1, LANES):
                tot = tot + plsc.load_gather(part_v, [lane_iota * LANES + l])
            out_v[pl.ds(h * half + g * LANES, LANES)] = (
                1.0 / (1.0 + jnp.exp(-tot)))
            return carry

        lax.fori_loop(0, half // LANES, group, 0)
        return 0

    lax.fori_loop(0, 2, do_half, 0)
    pltpu.sync_copy(out_v, out_hbm.at[pl.ds(base, B_PER_W)])


@jax.jit
def _mf(users, items, user_table, item_table):
    mesh = plsc.VectorSubcoreMesh(core_axis_name="c", subcore_axis_name="s")
    params = pltpu.CompilerParams(needs_layout_passes=False,
                                  use_tc_tiling_on_sc=True)
    utab3 = user_table.T.reshape(8, 8, NROWS)
    itab3 = item_table.T.reshape(8, 8, NROWS)

    gather = functools.partial(
        pl.kernel,
        mesh=mesh,
        compiler_params=params,
        out_type=(
            jax.ShapeDtypeStruct((BATCH + 8, EMBED_DIM), jnp.float32),
            jax.ShapeDtypeStruct((BATCH + 8, EMBED_DIM), jnp.float32),
        ),
        scratch_types=[
            pltpu.VMEM((BATCH,), jnp.int32),
            pltpu.VMEM((BATCH + LANES,), jnp.int32),
            pltpu.VMEM((BATCH + LANES,), jnp.int32),
            pltpu.VMEM((8, CHUNK), jnp.float32),
            pltpu.VMEM((8, CHUNK), jnp.float32),
            pltpu.VMEM((STAGE,), jnp.int32),
            pltpu.VMEM((STAGE, EMBED_DIM), jnp.float32),
            pltpu.SemaphoreType.DMA,
            pltpu.SemaphoreType.DMA,
            pltpu.SemaphoreType.DMA,
        ],
    )(_gather_body)
    ue, ie = gather(users, items, utab3, itab3)

    dot = functools.partial(
        pl.kernel,
        mesh=mesh,
        compiler_params=params,
        out_type=jax.ShapeDtypeStruct((BATCH,), jnp.float32),
        scratch_types=[
            pltpu.VMEM((B_PER_W // 2, EMBED_DIM), jnp.float32),
            pltpu.VMEM((B_PER_W // 2, EMBED_DIM), jnp.float32),
            pltpu.VMEM((B_PER_W,), jnp.float32),
            pltpu.VMEM((LANES * LANES,), jnp.float32),
            pltpu.SemaphoreType.DMA,
        ],
    )(_dot_body)
    return dot(ue, ie)


def kernel(users, items, user_table, item_table):
    return _mf(users.astype(jnp.int32), items.astype(jnp.int32),
               user_table, item_table)


# speedup vs baseline: 1.0041x; 1.0041x over previous
import functools

import jax
import jax.numpy as jnp
from jax import lax
from jax.experimental import pallas as pl
from jax.experimental.pallas import tpu as pltpu
from jax.experimental.pallas import tpu_sc as plsc

BATCH = 16384
NROWS = 1000000
EMBED_DIM = 64
LANES = 16
NUM_WORKERS = 32
STRIPE = 32768
CHUNK = 4096
NCHUNK = STRIPE // CHUNK
STAGE = 112
B_PER_W = BATCH // NUM_WORKERS
NVEC = BATCH // LANES

_MAIN_END = 983040
_W30_END = 999936
_TAIL = NROWS - _W30_END


def _gather_side(idx_hbm, idx_v, tab3, emb_hbm, lo, n_chunks, tail512,
                 tail64, slist_v, clist_v, chunk_v, chunk2_v, off_v,
                 stage_v, dsem, wsem):
    iota = lax.iota(jnp.int32, LANES)
    pltpu.sync_copy(idx_hbm, idx_v)
    hi = jnp.where(tail64, NROWS,
                   jnp.where(tail512, _W30_END, lo + n_chunks * CHUNK))

    def scan_body(v, cnt):
        vec = idx_v[pl.ds(v * LANES, LANES)]
        m = (vec >= lo) & (vec < hi)
        plsc.store_compressed(slist_v.at[pl.ds(cnt, LANES)], iota + v * LANES, mask=m)
        return cnt + plsc.all_reduce_population_count(m)[0]

    scnt = lax.fori_loop(0, NVEC, scan_body, jnp.int32(0))

    def process_chunk(cb, cb_next, width, self_prime, is_last):
        def cscan(j, cc):
            posv = slist_v[pl.ds(j * LANES, LANES)]
            mv = (iota + j * LANES) < scnt
            uv = plsc.load_gather(idx_v, [posv], mask=mv)
            m = mv & (uv >= cb) & (uv < cb + width)
            plsc.store_compressed(clist_v.at[pl.ds(cc, LANES)], posv, mask=m)
            return cc + plsc.all_reduce_population_count(m)[0]

        ccnt = lax.fori_loop(0, (scnt + LANES - 1) // LANES, cscan,
                             jnp.int32(0))
        nbatch = jnp.maximum((ccnt + STAGE - 1) // STAGE, 1)

        def issue_at(cbx, s, buf):
            return pltpu.async_copy(
                tab3.at[s, :, pl.ds(cbx, width)],
                buf.at[:, pl.ds(0, width)], dsem)

        def wait_one():
            pltpu.make_async_copy(
                tab3.at[0, :, pl.ds(cb, width)],
                chunk_v.at[:, pl.ds(0, width)], dsem).wait()

        if self_prime:
            issue_at(cb, 0, chunk_v)

        def batch_body(b, _):
            bbase = b * STAGE
            cnt_b = jnp.minimum(ccnt - bbase, STAGE)
            ngroups = (cnt_b + LANES - 1) // LANES

            def ogroup(jj, _):
                ev = iota + jj * LANES
                mb = ev < cnt_b
                posv = clist_v[pl.ds(bbase + jj * LANES, LANES)]
                uv = plsc.load_gather(idx_v, [posv], mask=mb)
                off_v[pl.ds(jj * LANES, LANES)] = uv - cb
                return 0

            lax.fori_loop(0, ngroups, ogroup, 0)

            def gather_slab(s, buf):
                def ggroup(jj, _):
                    ev = iota + jj * LANES
                    mb = ev < cnt_b
                    off = off_v[pl.ds(jj * LANES, LANES)]
                    for k in range(8):
                        vals = plsc.load_gather(
                            buf, [jnp.full((LANES,), k, jnp.int32), off],
                            mask=mb)
                        plsc.store_scatter(
                            stage_v,
                            [ev, jnp.broadcast_to(s * 8 + k, (LANES,))],
                            vals, mask=mb)
                    return 0

                lax.fori_loop(0, ngroups, ggroup, 0)

            last_batch = b == nbatch - 1

            def pair_body(t, _):
                issue_at(cb, 2 * t + 1, chunk2_v)
                wait_one()
                gather_slab(2 * t, chunk_v)

                @pl.when(t < 3)
                def _pref():
                    issue_at(cb, 2 * t + 2, chunk_v)

                @pl.when((t == 3) & ~(last_batch & is_last))
                def _pref_next():
                    issue_at(jnp.where(last_batch, cb_next, cb), 0, chunk_v)

                wait_one()
                gather_slab(2 * t + 1, chunk2_v)
                return 0

            lax.fori_loop(0, 4, pair_body, 0)

            def fgroup(jj, _):
                ev = iota + jj * LANES
                posv = clist_v[pl.ds(bbase + jj * LANES, LANES)]
                posv = jnp.where(ev < cnt_b, posv, BATCH)
                for l in range(LANES):
                    pltpu.async_copy(
                        stage_v.at[pl.ds(jj * LANES + l, 1), :],
                        emb_hbm.at[pl.ds(posv[l], 1), :], wsem)
                return 0

            lax.fori_loop(0, ngroups, fgroup, 0)

            def fdrain(i, _):
                pltpu.make_async_copy(
                    stage_v.at[pl.ds(0, 1), :],
                    emb_hbm.at[pl.ds(BATCH, 1), :], wsem).wait()
                return 0

            lax.fori_loop(0, ngroups * LANES, fdrain, 0)
            return 0

        lax.fori_loop(0, nbatch, batch_body, 0)

    @pl.when(n_chunks > 0)
    def _prime():
        pltpu.async_copy(
            tab3.at[0, :, pl.ds(pl.multiple_of(lo, CHUNK), CHUNK)],
            chunk_v.at[:, pl.ds(0, CHUNK)], dsem)

    def chunk_loop(c, _):
        cb = pl.multiple_of(lo + c * CHUNK, CHUNK)
        cb_next = pl.multiple_of(lo + (c + 1) * CHUNK, CHUNK)
        process_chunk(cb, cb_next, CHUNK, False, c == n_chunks - 1)
        return 0

    lax.fori_loop(0, n_chunks, chunk_loop, 0)

    @pl.when(tail512)
    def _t512():
        cb = pl.multiple_of(jnp.int32(_MAIN_END + 4 * CHUNK), 512)
        process_chunk(cb, cb, 512, True, jnp.bool_(True))

    @pl.when(tail64)
    def _t64():
        cb = pl.multiple_of(jnp.int32(_W30_END), 64)
        process_chunk(cb, cb, _TAIL, True, jnp.bool_(True))


def _gather_body(users_hbm, items_hbm, utab3, itab3, ue_hbm, ie_hbm,
                 idx_v, slist_v, clist_v, chunk_v, chunk2_v, off_v,
                 stage_v, isem, dsem, wsem):
    wid = lax.axis_index("s") * 2 + lax.axis_index("c")

    lo = wid * STRIPE
    n_chunks = jnp.where(wid < 30, NCHUNK, jnp.where(wid == 30, 4, 0))
    tail512 = wid == 30
    tail64 = wid == 31
    lo = jnp.where(tail64, _W30_END, lo)

    _gather_side(users_hbm, idx_v, utab3, ue_hbm, lo, n_chunks, tail512,
                 tail64, slist_v, clist_v, chunk_v, chunk2_v, off_v,
                 stage_v, dsem, wsem)
    _gather_side(items_hbm, idx_v, itab3, ie_hbm, lo, n_chunks, tail512,
                 tail64, slist_v, clist_v, chunk_v, chunk2_v, off_v,
                 stage_v, dsem, wsem)


def _dot_body(ue_hbm, ie_hbm, out_hbm, ubuf_v, ibuf_v, out_v, part_v, sem):
    wid = lax.axis_index("s") * 2 + lax.axis_index("c")
    base = wid * B_PER_W
    half = B_PER_W // 2
    lane_iota = lax.iota(jnp.int32, LANES)

    def do_half(h, _):
        hbase = base + h * half
        cp1 = pltpu.async_copy(ue_hbm.at[pl.ds(hbase, half), :], ubuf_v, sem)
        cp1.wait()
        cp2 = pltpu.async_copy(ie_hbm.at[pl.ds(hbase, half), :], ibuf_v, sem)
        cp2.wait()

        def group(g, carry):
            rowbase = g * LANES
            for r in range(LANES):
                row = rowbase + r
                acc = (ubuf_v[row, pl.ds(0, LANES)]
                       * ibuf_v[row, pl.ds(0, LANES)])
                for k in range(1, EMBED_DIM // LANES):
                    acc = acc + (ubuf_v[row, pl.ds(k * LANES, LANES)]
                                 * ibuf_v[row, pl.ds(k * LANES, LANES)])
                part_v[pl.ds(r * LANES, LANES)] = acc
            tot = plsc.load_gather(part_v, [lane_iota * LANES])
            for l in range(1, LANES):
                tot = tot + plsc.load_gather(part_v, [lane_iota * LANES + l])
            out_v[pl.ds(h * half + g * LANES, LANES)] = (
                1.0 / (1.0 + jnp.exp(-tot)))
            return carry

        lax.fori_loop(0, half // LANES, group, 0)
        return 0

    lax.fori_loop(0, 2, do_half, 0)
    pltpu.sync_copy(out_v, out_hbm.at[pl.ds(base, B_PER_W)])


@jax.jit
def _mf(users, items, user_table, item_table):
    mesh = plsc.VectorSubcoreMesh(core_axis_name="c", subcore_axis_name="s")
    params = pltpu.CompilerParams(needs_layout_passes=False,
                                  use_tc_tiling_on_sc=True)
    utab3 = user_table.T.reshape(8, 8, NROWS)
    itab3 = item_table.T.reshape(8, 8, NROWS)

    gather = functools.partial(
        pl.kernel,
        mesh=mesh,
        compiler_params=params,
        out_type=(
            jax.ShapeDtypeStruct((BATCH + 8, EMBED_DIM), jnp.float32),
            jax.ShapeDtypeStruct((BATCH + 8, EMBED_DIM), jnp.float32),
        ),
        scratch_types=[
            pltpu.VMEM((BATCH,), jnp.int32),
            pltpu.VMEM((BATCH + LANES,), jnp.int32),
            pltpu.VMEM((BATCH + LANES,), jnp.int32),
            pltpu.VMEM((8, CHUNK), jnp.float32),
            pltpu.VMEM((8, CHUNK), jnp.float32),
            pltpu.VMEM((STAGE,), jnp.int32),
            pltpu.VMEM((STAGE, EMBED_DIM), jnp.float32),
            pltpu.SemaphoreType.DMA,
            pltpu.SemaphoreType.DMA,
            pltpu.SemaphoreType.DMA,
        ],
    )(_gather_body)
    ue, ie = gather(users, items, utab3, itab3)

    dot = functools.partial(
        pl.kernel,
        mesh=mesh,
        compiler_params=params,
        out_type=jax.ShapeDtypeStruct((BATCH,), jnp.float32),
        scratch_types=[
            pltpu.VMEM((B_PER_W // 2, EMBED_DIM), jnp.float32),
            pltpu.VMEM((B_PER_W // 2, EMBED_DIM), jnp.float32),
            pltpu.VMEM((B_PER_W,), jnp.float32),
            pltpu.VMEM((LANES * LANES,), jnp.float32),
            pltpu.SemaphoreType.DMA,
        ],
    )(_dot_body)
    return dot(ue, ie)


def kernel(users, items, user_table, item_table):
    return _mf(users.astype(jnp.int32), items.astype(jnp.int32),
               user_table, item_table)
